# baseline (speedup 1.0000x reference)
import jax
import jax.numpy as jnp
from jax import lax
from jax.experimental import pallas as pl
from jax.experimental.pallas import tpu as pltpu
from jax.experimental.pallas import tpu_sc as plsc

N_ATOMS = 10000
N_BONDS = 320000
MAX_NB = 32
HID = 128
NLC = 8

NC, NS = 2, 16
NW = NC * NS

BA = 8
NBA = 40
APW = BA * NBA
PA = NW * APW

BPW = N_BONDS // NW
BB = 80
NBB = BPW // BB

MM_BLK = 2000


def _atom_body(a2b_hbm, ma_hbm, mbond_hbm, out_hbm,
               idx_all, rows0, rows1, acc, s0, s1):
    wid = lax.axis_index("s") * NC + lax.axis_index("c")
    abase = wid * APW
    pltpu.sync_copy(a2b_hbm.at[wid], idx_all)
    pltpu.sync_copy(ma_hbm.at[pl.ds(abase, APW)], acc)

    def gath(b, buf, sem):
        bc = jnp.minimum(b, NBA - 1)
        pltpu.async_copy(mbond_hbm.at[idx_all.at[2 * bc]],
                         buf.at[pl.ds(0, 128)], sem)
        pltpu.async_copy(mbond_hbm.at[idx_all.at[2 * bc + 1]],
                         buf.at[pl.ds(128, 128)], sem)

    def waitg(buf, sem):
        pltpu.make_async_copy(mbond_hbm.at[idx_all.at[0]],
                              buf.at[pl.ds(0, 128)], sem).wait()
        pltpu.make_async_copy(mbond_hbm.at[idx_all.at[0]],
                              buf.at[pl.ds(128, 128)], sem).wait()

    def compute(b, buf):
        def atom(i, carry2):
            r0 = i * MAX_NB
            v0 = [buf[r0, pl.ds(16 * c, 16)] for c in range(NLC)]

            def red(j, a):
                vs = [buf[r0 + j, pl.ds(16 * c, 16)] for c in range(NLC)]
                s = [a[c] + vs[c] for c in range(NLC)]
                m = [jnp.maximum(a[NLC + c], vs[c]) for c in range(NLC)]
                return tuple(s + m)

            a = lax.fori_loop(1, MAX_NB, red, tuple(v0 + v0), unroll=2)
            row = b * BA + i
            for c in range(NLC):
                sl = pl.ds(16 * c, 16)
                acc[row, sl] = acc[row, sl] + a[c] * a[NLC + c]
            return carry2

        lax.fori_loop(0, BA, atom, 0)

    gath(0, rows0, s0)
    gath(1, rows1, s1)

    def pair(t, carry):
        b0 = 2 * t
        waitg(rows0, s0)
        compute(b0, rows0)
        gath(b0 + 2, rows0, s0)
        waitg(rows1, s1)
        compute(b0 + 1, rows1)
        gath(b0 + 3, rows1, s1)
        return carry

    lax.fori_loop(0, NBA // 2, pair, 0)
    waitg(rows0, s0)
    waitg(rows1, s1)
    pltpu.sync_copy(acc, out_hbm.at[pl.ds(abase, APW)])


_atom_kernel = pl.kernel(
    _atom_body,
    out_type=jax.ShapeDtypeStruct((PA, HID), jnp.float32),
    mesh=plsc.VectorSubcoreMesh(core_axis_name="c", subcore_axis_name="s"),
    scratch_types=[
        pltpu.VMEM((2 * NBA, 128), jnp.int32),
        pltpu.VMEM((BA * MAX_NB, HID), jnp.float32),
        pltpu.VMEM((BA * MAX_NB, HID), jnp.float32),
        pltpu.VMEM((APW, HID), jnp.float32),
        pltpu.SemaphoreType.DMA,
        pltpu.SemaphoreType.DMA,
    ],
)


def _bond_body(b2a_hbm, b2revb_hbm, manew_hbm, mbond_hbm, g_hbm,
               idx_a, idx_r, ra0, ra1, rr0, rr1, ob0, ob1,
               sg0, sg1, so0, so1):
    wid = lax.axis_index("s") * NC + lax.axis_index("c")
    bbase = wid * BPW
    pltpu.sync_copy(b2a_hbm.at[wid], idx_a)
    pltpu.sync_copy(b2revb_hbm.at[wid], idx_r)

    def gath(k, ra, rr, sg):
        kc = jnp.minimum(k, NBB - 1)
        pltpu.async_copy(manew_hbm.at[idx_a.at[kc]], ra, sg)
        pltpu.async_copy(mbond_hbm.at[idx_r.at[kc]], rr, sg)

    def waitg(ra, rr, sg):
        pltpu.make_async_copy(manew_hbm.at[idx_a.at[0]], ra, sg).wait()
        pltpu.make_async_copy(mbond_hbm.at[idx_r.at[0]], rr, sg).wait()

    def waitst(ob, so):
        pltpu.make_async_copy(ob, g_hbm.at[pl.ds(bbase, BB)], so).wait()

    def compute_store(k, ra, rr, ob, so):
        kc = jnp.minimum(k, NBB - 1)

        def row(i, carry2):
            for c in range(NLC):
                sl = pl.ds(16 * c, 16)
                ob[i, sl] = ra[i, sl] - rr[i, sl]
            return carry2

        lax.fori_loop(0, BB, row, 0, unroll=2)
        pltpu.async_copy(ob, g_hbm.at[pl.ds(bbase + BB * kc, BB)], so)

    gath(0, ra0, rr0, sg0)
    gath(1, ra1, rr1, sg1)

    def pair(t, carry):
        b0 = 2 * t
        waitg(ra0, rr0, sg0)

        @pl.when(t > 0)
        def _():
            waitst(ob0, so0)

        compute_store(b0, ra0, rr0, ob0, so0)
        gath(b0 + 2, ra0, rr0, sg0)
        waitg(ra1, rr1, sg1)

        @pl.when(t > 0)
        def _():
            waitst(ob1, so1)

        compute_store(b0 + 1, ra1, rr1, ob1, so1)
        gath(b0 + 3, ra1, rr1, sg1)
        return carry

    lax.fori_loop(0, (NBB + 1) // 2, pair, 0)
    waitg(ra0, rr0, sg0)
    waitg(ra1, rr1, sg1)
    waitst(ob0, so0)
    waitst(ob1, so1)


_bond_kernel = pl.kernel(
    _bond_body,
    out_type=jax.ShapeDtypeStruct((N_BONDS, HID), jnp.float32),
    mesh=plsc.VectorSubcoreMesh(core_axis_name="c", subcore_axis_name="s"),
    scratch_types=[
        pltpu.VMEM((NBB, BB), jnp.int32),
        pltpu.VMEM((NBB, BB), jnp.int32),
        pltpu.VMEM((BB, HID), jnp.float32),
        pltpu.VMEM((BB, HID), jnp.float32),
        pltpu.VMEM((BB, HID), jnp.float32),
        pltpu.VMEM((BB, HID), jnp.float32),
        pltpu.VMEM((BB, HID), jnp.float32),
        pltpu.VMEM((BB, HID), jnp.float32),
        pltpu.SemaphoreType.DMA,
        pltpu.SemaphoreType.DMA,
        pltpu.SemaphoreType.DMA,
        pltpu.SemaphoreType.DMA,
    ],
)


def _mm_body(g_ref, in_ref, wt_ref, b_ref, o_ref):
    mm = jnp.dot(g_ref[...], wt_ref[...], preferred_element_type=jnp.float32)
    o_ref[...] = jnp.maximum(in_ref[...] + mm + b_ref[...], 0.0)


def _linear_relu(g, input_bond, wt, b2d):
    grid = N_BONDS // MM_BLK
    return pl.pallas_call(
        _mm_body,
        grid=(grid,),
        in_specs=[
            pl.BlockSpec((MM_BLK, HID), lambda i: (i, 0)),
            pl.BlockSpec((MM_BLK, HID), lambda i: (i, 0)),
            pl.BlockSpec((HID, HID), lambda i: (0, 0)),
            pl.BlockSpec((1, HID), lambda i: (0, 0)),
        ],
        out_specs=pl.BlockSpec((MM_BLK, HID), lambda i: (i, 0)),
        out_shape=jax.ShapeDtypeStruct((N_BONDS, HID), jnp.float32),
    )(g, input_bond, wt, b2d)


def kernel(message_atom, message_bond, a2b, b2a, b2revb, input_bond, W_bond, b_bond):
    a2b = a2b.astype(jnp.int32)
    b2a = b2a.astype(jnp.int32)
    b2revb = b2revb.astype(jnp.int32)

    ma_pad = jnp.pad(message_atom, ((0, PA - N_ATOMS), (0, 0)))
    a2b_pad = jnp.pad(a2b.reshape(-1), (0, (PA - N_ATOMS) * MAX_NB))
    a2b_pad = a2b_pad.reshape(NW, 2 * NBA, 128)
    b2a_r = b2a.reshape(NW, NBB, BB)
    b2revb_r = b2revb.reshape(NW, NBB, BB)

    manew_pad = _atom_kernel(a2b_pad, ma_pad, message_bond)
    g = _bond_kernel(b2a_r, b2revb_r, manew_pad, message_bond)
    mb = _linear_relu(g, input_bond, W_bond.T, b_bond.reshape(1, HID))
    return (manew_pad[:N_ATOMS], mb)


# speedup vs baseline: 1.0792x; 1.0792x over previous
import jax
import jax.numpy as jnp
from jax import lax
from jax.experimental import pallas as pl
from jax.experimental.pallas import tpu as pltpu
from jax.experimental.pallas import tpu_sc as plsc

N_ATOMS = 10000
N_BONDS = 320000
MAX_NB = 32
HID = 128
NLC = 8

NC, NS = 2, 16
NW = NC * NS

BA = 8
NBA = 40
APW = BA * NBA
PA = NW * APW

BPW = N_BONDS // NW
BB = 80
NBB = BPW // BB

MM_BLK = 2000


def _atom_body(a2b_hbm, ma_hbm, mbond_hbm, out_hbm,
               idx_all, r0a, r0b, r1a, r1b, acc, s0, s1):
    wid = lax.axis_index("s") * NC + lax.axis_index("c")
    abase = wid * APW
    pltpu.sync_copy(a2b_hbm.at[wid], idx_all)
    pltpu.sync_copy(ma_hbm.at[pl.ds(abase, APW)], acc)

    def gath(b, bufa, bufb, sem):
        bc = jnp.minimum(b, NBA - 1)
        pltpu.async_copy(mbond_hbm.at[idx_all.at[2 * bc]], bufa, sem)
        pltpu.async_copy(mbond_hbm.at[idx_all.at[2 * bc + 1]], bufb, sem)

    def waitg(bufa, bufb, sem):
        pltpu.make_async_copy(mbond_hbm.at[idx_all.at[0]], bufa, sem).wait()
        pltpu.make_async_copy(mbond_hbm.at[idx_all.at[0]], bufb, sem).wait()

    def compute(b, bufa, bufb):
        def half(buf, half_idx):
            def atom(i, carry2):
                r0 = i * MAX_NB
                v0 = [buf[r0, pl.ds(16 * c, 16)] for c in range(NLC)]

                def red(j, a):
                    vs = [buf[r0 + j, pl.ds(16 * c, 16)] for c in range(NLC)]
                    s = [a[c] + vs[c] for c in range(NLC)]
                    m = [jnp.maximum(a[NLC + c], vs[c]) for c in range(NLC)]
                    return tuple(s + m)

                a = lax.fori_loop(1, MAX_NB, red, tuple(v0 + v0), unroll=2)
                row = b * BA + half_idx * (BA // 2) + i
                for c in range(NLC):
                    sl = pl.ds(16 * c, 16)
                    acc[row, sl] = acc[row, sl] + a[c] * a[NLC + c]
                return carry2

            lax.fori_loop(0, BA // 2, atom, 0)

        half(bufa, 0)
        half(bufb, 1)

    gath(0, r0a, r0b, s0)
    gath(1, r1a, r1b, s1)

    def pair(t, carry):
        b0 = 2 * t
        waitg(r0a, r0b, s0)
        compute(b0, r0a, r0b)
        gath(b0 + 2, r0a, r0b, s0)
        waitg(r1a, r1b, s1)
        compute(b0 + 1, r1a, r1b)
        gath(b0 + 3, r1a, r1b, s1)
        return carry

    lax.fori_loop(0, NBA // 2, pair, 0)
    waitg(r0a, r0b, s0)
    waitg(r1a, r1b, s1)
    pltpu.sync_copy(acc, out_hbm.at[pl.ds(abase, APW)])


_atom_kernel = pl.kernel(
    _atom_body,
    out_type=jax.ShapeDtypeStruct((PA, HID), jnp.float32),
    mesh=plsc.VectorSubcoreMesh(core_axis_name="c", subcore_axis_name="s"),
    scratch_types=[
        pltpu.VMEM((2 * NBA, 128), jnp.int32),
        pltpu.VMEM((128, HID), jnp.float32),
        pltpu.VMEM((128, HID), jnp.float32),
        pltpu.VMEM((128, HID), jnp.float32),
        pltpu.VMEM((128, HID), jnp.float32),
        pltpu.VMEM((APW, HID), jnp.float32),
        pltpu.SemaphoreType.DMA,
        pltpu.SemaphoreType.DMA,
    ],
)


def _bond_body(b2a_hbm, b2revb_hbm, manew_hbm, mbond_hbm, g_hbm,
               idx_a, idx_r, rows_a, rows_r, sema, semr):
    wid = lax.axis_index("s") * NC + lax.axis_index("c")
    bbase = wid * BPW
    pltpu.sync_copy(b2a_hbm.at[wid], idx_a)
    pltpu.sync_copy(b2revb_hbm.at[wid], idx_r)

    def batch(k, carry):
        ca = pltpu.async_copy(manew_hbm.at[idx_a.at[k]], rows_a, sema)
        cr = pltpu.async_copy(mbond_hbm.at[idx_r.at[k]], rows_r, semr)
        ca.wait()
        cr.wait()

        def row(i, carry2):
            for c in range(NLC):
                sl = pl.ds(16 * c, 16)
                rows_a[i, sl] = rows_a[i, sl] - rows_r[i, sl]
            return carry2

        lax.fori_loop(0, BB, row, 0)
        pltpu.sync_copy(rows_a, g_hbm.at[pl.ds(bbase + BB * k, BB)])
        return carry

    lax.fori_loop(0, NBB, batch, 0)


_bond_kernel = pl.kernel(
    _bond_body,
    out_type=jax.ShapeDtypeStruct((N_BONDS, HID), jnp.float32),
    mesh=plsc.VectorSubcoreMesh(core_axis_name="c", subcore_axis_name="s"),
    scratch_types=[
        pltpu.VMEM((NBB, BB), jnp.int32),
        pltpu.VMEM((NBB, BB), jnp.int32),
        pltpu.VMEM((BB, HID), jnp.float32),
        pltpu.VMEM((BB, HID), jnp.float32),
        pltpu.SemaphoreType.DMA,
        pltpu.SemaphoreType.DMA,
    ],
)


def _mm_body(g_ref, in_ref, wt_ref, b_ref, o_ref):
    mm = jnp.dot(g_ref[...], wt_ref[...], preferred_element_type=jnp.float32)
    o_ref[...] = jnp.maximum(in_ref[...] + mm + b_ref[...], 0.0)


def _linear_relu(g, input_bond, wt, b2d):
    grid = N_BONDS // MM_BLK
    return pl.pallas_call(
        _mm_body,
        grid=(grid,),
        in_specs=[
            pl.BlockSpec((MM_BLK, HID), lambda i: (i, 0)),
            pl.BlockSpec((MM_BLK, HID), lambda i: (i, 0)),
            pl.BlockSpec((HID, HID), lambda i: (0, 0)),
            pl.BlockSpec((1, HID), lambda i: (0, 0)),
        ],
        out_specs=pl.BlockSpec((MM_BLK, HID), lambda i: (i, 0)),
        out_shape=jax.ShapeDtypeStruct((N_BONDS, HID), jnp.float32),
    )(g, input_bond, wt, b2d)


def kernel(message_atom, message_bond, a2b, b2a, b2revb, input_bond, W_bond, b_bond):
    a2b = a2b.astype(jnp.int32)
    b2a = b2a.astype(jnp.int32)
    b2revb = b2revb.astype(jnp.int32)

    ma_pad = jnp.pad(message_atom, ((0, PA - N_ATOMS), (0, 0)))
    a2b_pad = jnp.pad(a2b.reshape(-1), (0, (PA - N_ATOMS) * MAX_NB))
    a2b_pad = a2b_pad.reshape(NW, 2 * NBA, 128)
    b2a_r = b2a.reshape(NW, NBB, BB)
    b2revb_r = b2revb.reshape(NW, NBB, BB)

    manew_pad = _atom_kernel(a2b_pad, ma_pad, message_bond)
    g = _bond_kernel(b2a_r, b2revb_r, manew_pad, message_bond)
    mb = _linear_relu(g, input_bond, W_bond.T, b_bond.reshape(1, HID))
    return (manew_pad[:N_ATOMS], mb)


# speedup vs baseline: 1.8231x; 1.6894x over previous
import jax
import jax.numpy as jnp
from jax import lax
from jax.experimental import pallas as pl
from jax.experimental.pallas import tpu as pltpu
from jax.experimental.pallas import tpu_sc as plsc

N_ATOMS = 10000
N_BONDS = 320000
MAX_NB = 32
HID = 128
NLC = 8

NC, NS = 2, 16
NW = NC * NS

BA = 8
NBA = 40
APW = BA * NBA
PA = NW * APW

BPW = N_BONDS // NW
BB = 80
NBB = BPW // BB

MM_BLK = 2000


def _atom_body(a2b_hbm, ma_hbm, mbond_hbm, out_hbm,
               idx_all, r0a, r0b, r1a, r1b, acc, s0, s1):
    wid = lax.axis_index("s") * NC + lax.axis_index("c")
    abase = wid * APW
    pltpu.sync_copy(a2b_hbm.at[wid], idx_all)
    pltpu.sync_copy(ma_hbm.at[pl.ds(abase, APW)], acc)

    def gath(b, bufa, bufb, sem):
        bc = jnp.minimum(b, NBA - 1)
        pltpu.async_copy(mbond_hbm.at[idx_all.at[2 * bc]], bufa, sem)
        pltpu.async_copy(mbond_hbm.at[idx_all.at[2 * bc + 1]], bufb, sem)

    def waitg(bufa, bufb, sem):
        pltpu.make_async_copy(mbond_hbm.at[idx_all.at[0]], bufa, sem).wait()
        pltpu.make_async_copy(mbond_hbm.at[idx_all.at[0]], bufb, sem).wait()

    def compute(b, bufa, bufb):
        def half(buf, half_idx):
            def atom(i, carry2):
                r0 = i * MAX_NB
                v0 = [buf[r0, pl.ds(16 * c, 16)] for c in range(NLC)]

                def red(j, a):
                    vs = [buf[r0 + j, pl.ds(16 * c, 16)] for c in range(NLC)]
                    s = [a[c] + vs[c] for c in range(NLC)]
                    m = [jnp.maximum(a[NLC + c], vs[c]) for c in range(NLC)]
                    return tuple(s + m)

                a = lax.fori_loop(1, MAX_NB, red, tuple(v0 + v0), unroll=2)
                row = b * BA + half_idx * (BA // 2) + i
                for c in range(NLC):
                    sl = pl.ds(16 * c, 16)
                    acc[row, sl] = acc[row, sl] + a[c] * a[NLC + c]
                return carry2

            lax.fori_loop(0, BA // 2, atom, 0)

        half(bufa, 0)
        half(bufb, 1)

    gath(0, r0a, r0b, s0)
    gath(1, r1a, r1b, s1)

    def pair(t, carry):
        b0 = 2 * t
        waitg(r0a, r0b, s0)
        compute(b0, r0a, r0b)
        gath(b0 + 2, r0a, r0b, s0)
        waitg(r1a, r1b, s1)
        compute(b0 + 1, r1a, r1b)
        gath(b0 + 3, r1a, r1b, s1)
        return carry

    lax.fori_loop(0, NBA // 2, pair, 0)
    waitg(r0a, r0b, s0)
    waitg(r1a, r1b, s1)
    pltpu.sync_copy(acc, out_hbm.at[pl.ds(abase, APW)])


_atom_kernel = pl.kernel(
    _atom_body,
    out_type=jax.ShapeDtypeStruct((PA, HID), jnp.float32),
    mesh=plsc.VectorSubcoreMesh(core_axis_name="c", subcore_axis_name="s"),
    scratch_types=[
        pltpu.VMEM((2 * NBA, 128), jnp.int32),
        pltpu.VMEM((128, HID), jnp.float32),
        pltpu.VMEM((128, HID), jnp.float32),
        pltpu.VMEM((128, HID), jnp.float32),
        pltpu.VMEM((128, HID), jnp.float32),
        pltpu.VMEM((APW, HID), jnp.float32),
        pltpu.SemaphoreType.DMA,
        pltpu.SemaphoreType.DMA,
    ],
)


def _bond_body(b2a_hbm, b2revb_hbm, manew_hbm, mbond_hbm, g_hbm,
               idx_a, idx_r, rows_a, rows_r, sema, semr):
    wid = lax.axis_index("s") * NC + lax.axis_index("c")
    bbase = wid * BPW
    pltpu.sync_copy(b2a_hbm.at[wid], idx_a)
    pltpu.sync_copy(b2revb_hbm.at[wid], idx_r)

    def batch(k, carry):
        ca = pltpu.async_copy(manew_hbm.at[idx_a.at[k]], rows_a, sema)
        cr = pltpu.async_copy(mbond_hbm.at[idx_r.at[k]], rows_r, semr)
        ca.wait()
        cr.wait()

        def row(i, carry2):
            for c in range(NLC):
                sl = pl.ds(16 * c, 16)
                rows_a[i, sl] = rows_a[i, sl] - rows_r[i, sl]
            return carry2

        lax.fori_loop(0, BB, row, 0)
        pltpu.sync_copy(rows_a, g_hbm.at[pl.ds(bbase + BB * k, BB)])
        return carry

    lax.fori_loop(0, NBB, batch, 0)


_bond_kernel = pl.kernel(
    _bond_body,
    out_type=jax.ShapeDtypeStruct((N_BONDS, HID), jnp.float32),
    mesh=plsc.VectorSubcoreMesh(core_axis_name="c", subcore_axis_name="s"),
    scratch_types=[
        pltpu.VMEM((NBB, BB), jnp.int32),
        pltpu.VMEM((NBB, BB), jnp.int32),
        pltpu.VMEM((BB, HID), jnp.float32),
        pltpu.VMEM((BB, HID), jnp.float32),
        pltpu.SemaphoreType.DMA,
        pltpu.SemaphoreType.DMA,
    ],
)


def _mm_body(g_ref, in_ref, wt_ref, b_ref, o_ref):
    mm = jnp.dot(g_ref[...], wt_ref[...], preferred_element_type=jnp.float32)
    o_ref[...] = jnp.maximum(in_ref[...] + mm + b_ref[...], 0.0)


def _linear_relu(g, input_bond, wt, b2d):
    grid = N_BONDS // MM_BLK
    return pl.pallas_call(
        _mm_body,
        grid=(grid,),
        in_specs=[
            pl.BlockSpec((MM_BLK, HID), lambda i: (i, 0)),
            pl.BlockSpec((MM_BLK, HID), lambda i: (i, 0)),
            pl.BlockSpec((HID, HID), lambda i: (0, 0)),
            pl.BlockSpec((1, HID), lambda i: (0, 0)),
        ],
        out_specs=pl.BlockSpec((MM_BLK, HID), lambda i: (i, 0)),
        out_shape=jax.ShapeDtypeStruct((N_BONDS, HID), jnp.float32),
    )(g, input_bond, wt, b2d)


def kernel(message_atom, message_bond, a2b, b2a, b2revb, input_bond, W_bond, b_bond):
    a2b = a2b.astype(jnp.int32)
    b2a = b2a.astype(jnp.int32)
    b2revb = b2revb.astype(jnp.int32)

    ma_pad = jnp.pad(message_atom, ((0, PA - N_ATOMS), (0, 0)))
    pad_idx = jnp.arange((PA - N_ATOMS) * MAX_NB, dtype=jnp.int32) % N_BONDS
    a2b_pad = jnp.concatenate([a2b.reshape(-1), pad_idx])
    a2b_pad = a2b_pad.reshape(NW, 2 * NBA, 128)
    b2a_r = b2a.reshape(NW, NBB, BB)
    b2revb_r = b2revb.reshape(NW, NBB, BB)

    manew_pad = _atom_kernel(a2b_pad, ma_pad, message_bond)
    g = _bond_kernel(b2a_r, b2revb_r, manew_pad, message_bond)
    mb = _linear_relu(g, input_bond, W_bond.T, b_bond.reshape(1, HID))
    return (manew_pad[:N_ATOMS], mb)


# speedup vs baseline: 2.1207x; 1.1633x over previous
import jax
import jax.numpy as jnp
from jax import lax
from jax.experimental import pallas as pl
from jax.experimental.pallas import tpu as pltpu
from jax.experimental.pallas import tpu_sc as plsc

N_ATOMS = 10000
N_BONDS = 320000
MAX_NB = 32
HID = 128
NLC = 8

NC, NS = 2, 16
NW = NC * NS

BA = 8
NBA = 40
APW = BA * NBA
PA = NW * APW

NCH = 5
CHB = N_BONDS // NCH
BPW = CHB // NW
BB = 80
NBB = BPW // BB

MM_BLK = 2000


def _atom_body(a2b_hbm, ma_hbm, mbond_hbm, out_hbm,
               idx_all, r0a, r0b, r1a, r1b, acc, s0, s1):
    wid = lax.axis_index("s") * NC + lax.axis_index("c")
    abase = wid * APW
    pltpu.sync_copy(a2b_hbm.at[wid], idx_all)
    pltpu.sync_copy(ma_hbm.at[pl.ds(abase, APW)], acc)

    def gath(b, bufa, bufb, sem):
        bc = jnp.minimum(b, NBA - 1)
        pltpu.async_copy(mbond_hbm.at[idx_all.at[2 * bc]], bufa, sem)
        pltpu.async_copy(mbond_hbm.at[idx_all.at[2 * bc + 1]], bufb, sem)

    def waitg(bufa, bufb, sem):
        pltpu.make_async_copy(mbond_hbm.at[idx_all.at[0]], bufa, sem).wait()
        pltpu.make_async_copy(mbond_hbm.at[idx_all.at[0]], bufb, sem).wait()

    def compute(b, bufa, bufb):
        def half(buf, half_idx):
            def atom(i, carry2):
                r0 = i * MAX_NB
                v0 = [buf[r0, pl.ds(16 * c, 16)] for c in range(NLC)]

                def red(j, a):
                    vs = [buf[r0 + j, pl.ds(16 * c, 16)] for c in range(NLC)]
                    s = [a[c] + vs[c] for c in range(NLC)]
                    m = [jnp.maximum(a[NLC + c], vs[c]) for c in range(NLC)]
                    return tuple(s + m)

                a = lax.fori_loop(1, MAX_NB, red, tuple(v0 + v0), unroll=2)
                row = b * BA + half_idx * (BA // 2) + i
                for c in range(NLC):
                    sl = pl.ds(16 * c, 16)
                    acc[row, sl] = acc[row, sl] + a[c] * a[NLC + c]
                return carry2

            lax.fori_loop(0, BA // 2, atom, 0)

        half(bufa, 0)
        half(bufb, 1)

    gath(0, r0a, r0b, s0)
    gath(1, r1a, r1b, s1)

    def pair(t, carry):
        b0 = 2 * t
        waitg(r0a, r0b, s0)
        compute(b0, r0a, r0b)
        gath(b0 + 2, r0a, r0b, s0)
        waitg(r1a, r1b, s1)
        compute(b0 + 1, r1a, r1b)
        gath(b0 + 3, r1a, r1b, s1)
        return carry

    lax.fori_loop(0, NBA // 2, pair, 0)
    waitg(r0a, r0b, s0)
    waitg(r1a, r1b, s1)
    pltpu.sync_copy(acc, out_hbm.at[pl.ds(abase, APW)])


_atom_kernel = pl.kernel(
    _atom_body,
    out_type=jax.ShapeDtypeStruct((PA, HID), jnp.float32),
    mesh=plsc.VectorSubcoreMesh(core_axis_name="c", subcore_axis_name="s"),
    scratch_types=[
        pltpu.VMEM((2 * NBA, 128), jnp.int32),
        pltpu.VMEM((128, HID), jnp.float32),
        pltpu.VMEM((128, HID), jnp.float32),
        pltpu.VMEM((128, HID), jnp.float32),
        pltpu.VMEM((128, HID), jnp.float32),
        pltpu.VMEM((APW, HID), jnp.float32),
        pltpu.SemaphoreType.DMA,
        pltpu.SemaphoreType.DMA,
    ],
)


def _bond_body(b2a_hbm, b2revb_hbm, manew_hbm, mbond_hbm, g_hbm,
               idx_a, idx_r, rows_a, rows_r, sema, semr):
    wid = lax.axis_index("s") * NC + lax.axis_index("c")
    bbase = wid * BPW
    pltpu.sync_copy(b2a_hbm.at[wid], idx_a)
    pltpu.sync_copy(b2revb_hbm.at[wid], idx_r)

    def batch(k, carry):
        ca = pltpu.async_copy(manew_hbm.at[idx_a.at[k]], rows_a, sema)
        cr = pltpu.async_copy(mbond_hbm.at[idx_r.at[k]], rows_r, semr)
        ca.wait()
        cr.wait()

        def row(i, carry2):
            for c in range(NLC):
                sl = pl.ds(16 * c, 16)
                rows_a[i, sl] = rows_a[i, sl] - rows_r[i, sl]
            return carry2

        lax.fori_loop(0, BB, row, 0)
        pltpu.sync_copy(rows_a, g_hbm.at[pl.ds(bbase + BB * k, BB)])
        return carry

    lax.fori_loop(0, NBB, batch, 0)


_bond_kernel = pl.kernel(
    _bond_body,
    out_type=jax.ShapeDtypeStruct((CHB, HID), jnp.float32),
    mesh=plsc.VectorSubcoreMesh(core_axis_name="c", subcore_axis_name="s"),
    scratch_types=[
        pltpu.VMEM((NBB, BB), jnp.int32),
        pltpu.VMEM((NBB, BB), jnp.int32),
        pltpu.VMEM((BB, HID), jnp.float32),
        pltpu.VMEM((BB, HID), jnp.float32),
        pltpu.SemaphoreType.DMA,
        pltpu.SemaphoreType.DMA,
    ],
)


def _mm_body(g_ref, in_ref, wt_ref, b_ref, o_ref):
    mm = jnp.dot(g_ref[...], wt_ref[...], preferred_element_type=jnp.float32)
    o_ref[...] = jnp.maximum(in_ref[...] + mm + b_ref[...], 0.0)


def _mm_body_acc(m_ref, g_ref, in_ref, wt_ref, b_ref, o_ref):
    mm = jnp.dot(g_ref[...], wt_ref[...], preferred_element_type=jnp.float32)
    o_ref[...] = jnp.maximum(in_ref[...] + mm + b_ref[...], 0.0)


def _linear_relu_chunk(j, m, g_j, input_bond, wt, b2d):
    grid = CHB // MM_BLK
    gspec = pl.BlockSpec((MM_BLK, HID), lambda i: (i, 0))
    inspec = pl.BlockSpec((MM_BLK, HID), lambda i, j=j: (j * grid + i, 0))
    wspec = pl.BlockSpec((HID, HID), lambda i: (0, 0))
    bspec = pl.BlockSpec((1, HID), lambda i: (0, 0))
    outspec = pl.BlockSpec((MM_BLK, HID), lambda i, j=j: (j * grid + i, 0))
    out_shape = jax.ShapeDtypeStruct((N_BONDS, HID), jnp.float32)
    if j == 0:
        return pl.pallas_call(
            _mm_body,
            grid=(grid,),
            in_specs=[gspec, inspec, wspec, bspec],
            out_specs=outspec,
            out_shape=out_shape,
        )(g_j, input_bond, wt, b2d)
    mspec = pl.BlockSpec((8, HID), lambda i: (0, 0))
    return pl.pallas_call(
        _mm_body_acc,
        grid=(grid,),
        in_specs=[mspec, gspec, inspec, wspec, bspec],
        out_specs=outspec,
        out_shape=out_shape,
        input_output_aliases={0: 0},
    )(m, g_j, input_bond, wt, b2d)


def kernel(message_atom, message_bond, a2b, b2a, b2revb, input_bond, W_bond, b_bond):
    a2b = a2b.astype(jnp.int32)
    b2a = b2a.astype(jnp.int32)
    b2revb = b2revb.astype(jnp.int32)

    ma_pad = jnp.pad(message_atom, ((0, PA - N_ATOMS), (0, 0)))
    pad_idx = jnp.arange((PA - N_ATOMS) * MAX_NB, dtype=jnp.int32) % N_BONDS
    a2b_pad = jnp.concatenate([a2b.reshape(-1), pad_idx])
    a2b_pad = a2b_pad.reshape(NW, 2 * NBA, 128)
    b2a_r = b2a.reshape(NCH, NW, NBB, BB)
    b2revb_r = b2revb.reshape(NCH, NW, NBB, BB)

    manew_pad = _atom_kernel(a2b_pad, ma_pad, message_bond)
    wt = W_bond.T
    b2d = b_bond.reshape(1, HID)
    mb = None
    for j in range(NCH):
        g_j = _bond_kernel(b2a_r[j], b2revb_r[j], manew_pad, message_bond)
        mb = _linear_relu_chunk(j, mb, g_j, input_bond, wt, b2d)
    return (manew_pad[:N_ATOMS], mb)


# speedup vs baseline: 2.4992x; 1.1784x over previous
import jax
import jax.numpy as jnp
from jax import lax
from jax.experimental import pallas as pl
from jax.experimental.pallas import tpu as pltpu
from jax.experimental.pallas import tpu_sc as plsc

N_ATOMS = 10000
N_BONDS = 320000
MAX_NB = 32
HID = 128
NLC = 8

NC, NS = 2, 16
NW = NC * NS

BA = 8
NBA = 40
APW = BA * NBA
PA = NW * APW

NCH = 5
CHB = N_BONDS // NCH
BPW = CHB // NW
BB = 80
NBB = BPW // BB

MM_BLK = 2000


def _atom_body(a2b_hbm, ma_hbm, mbond_hbm, out_hbm,
               idx_all, r0a, r0b, r1a, r1b, acc, s0, s1):
    wid = lax.axis_index("s") * NC + lax.axis_index("c")
    abase = wid * APW
    pltpu.sync_copy(a2b_hbm.at[wid], idx_all)
    pltpu.sync_copy(ma_hbm.at[pl.ds(abase, APW)], acc)

    def gath(b, bufa, bufb, sem):
        bc = jnp.minimum(b, NBA - 1)
        pltpu.async_copy(mbond_hbm.at[idx_all.at[2 * bc]], bufa, sem)
        pltpu.async_copy(mbond_hbm.at[idx_all.at[2 * bc + 1]], bufb, sem)

    def waitg(bufa, bufb, sem):
        pltpu.make_async_copy(mbond_hbm.at[idx_all.at[0]], bufa, sem).wait()
        pltpu.make_async_copy(mbond_hbm.at[idx_all.at[0]], bufb, sem).wait()

    def compute(b, bufa, bufb):
        def half(buf, half_idx):
            def atom(i, carry2):
                r0 = i * MAX_NB
                v0 = [buf[r0, pl.ds(16 * c, 16)] for c in range(NLC)]

                def red(j, a):
                    vs = [buf[r0 + j, pl.ds(16 * c, 16)] for c in range(NLC)]
                    s = [a[c] + vs[c] for c in range(NLC)]
                    m = [jnp.maximum(a[NLC + c], vs[c]) for c in range(NLC)]
                    return tuple(s + m)

                a = lax.fori_loop(1, MAX_NB, red, tuple(v0 + v0), unroll=2)
                row = b * BA + half_idx * (BA // 2) + i
                for c in range(NLC):
                    sl = pl.ds(16 * c, 16)
                    acc[row, sl] = acc[row, sl] + a[c] * a[NLC + c]
                return carry2

            lax.fori_loop(0, BA // 2, atom, 0)

        half(bufa, 0)
        half(bufb, 1)

    gath(0, r0a, r0b, s0)
    gath(1, r1a, r1b, s1)

    def pair(t, carry):
        b0 = 2 * t
        waitg(r0a, r0b, s0)
        compute(b0, r0a, r0b)
        gath(b0 + 2, r0a, r0b, s0)
        waitg(r1a, r1b, s1)
        compute(b0 + 1, r1a, r1b)
        gath(b0 + 3, r1a, r1b, s1)
        return carry

    lax.fori_loop(0, NBA // 2, pair, 0)
    waitg(r0a, r0b, s0)
    waitg(r1a, r1b, s1)
    pltpu.sync_copy(acc, out_hbm.at[pl.ds(abase, APW)])


_atom_kernel = pl.kernel(
    _atom_body,
    out_type=jax.ShapeDtypeStruct((PA, HID), jnp.float32),
    mesh=plsc.VectorSubcoreMesh(core_axis_name="c", subcore_axis_name="s"),
    scratch_types=[
        pltpu.VMEM((2 * NBA, 128), jnp.int32),
        pltpu.VMEM((128, HID), jnp.float32),
        pltpu.VMEM((128, HID), jnp.float32),
        pltpu.VMEM((128, HID), jnp.float32),
        pltpu.VMEM((128, HID), jnp.float32),
        pltpu.VMEM((APW, HID), jnp.float32),
        pltpu.SemaphoreType.DMA,
        pltpu.SemaphoreType.DMA,
    ],
)


def _bond_body(b2a_hbm, b2revb_hbm, manew_hbm, mbond_hbm, g_hbm,
               idx_a, idx_r, ra0, rr0, ra1, rr1, ob0, ob1,
               sg0, sg1, so0, so1):
    wid = lax.axis_index("s") * NC + lax.axis_index("c")
    bbase = wid * BPW
    pltpu.sync_copy(b2a_hbm.at[wid], idx_a)
    pltpu.sync_copy(b2revb_hbm.at[wid], idx_r)

    def gath(k, ra, rr, sg):
        pltpu.async_copy(manew_hbm.at[idx_a.at[k]], ra, sg)
        pltpu.async_copy(mbond_hbm.at[idx_r.at[k]], rr, sg)

    def waitg(ra, rr, sg):
        pltpu.make_async_copy(manew_hbm.at[idx_a.at[0]], ra, sg).wait()
        pltpu.make_async_copy(mbond_hbm.at[idx_r.at[0]], rr, sg).wait()

    def waitst(ob, so):
        pltpu.make_async_copy(ob, g_hbm.at[pl.ds(bbase, BB)], so).wait()

    def comp_st(k, ra, rr, ob, so):
        def row(i, carry2):
            for c in range(NLC):
                sl = pl.ds(16 * c, 16)
                ob[i, sl] = ra[i, sl] - rr[i, sl]
            return carry2

        lax.fori_loop(0, BB, row, 0)
        pltpu.async_copy(ob, g_hbm.at[pl.ds(bbase + BB * k, BB)], so)

    gath(0, ra0, rr0, sg0)
    gath(1, ra1, rr1, sg1)
    waitg(ra0, rr0, sg0)
    comp_st(0, ra0, rr0, ob0, so0)
    gath(2, ra0, rr0, sg0)
    waitg(ra1, rr1, sg1)
    comp_st(1, ra1, rr1, ob1, so1)
    gath(3, ra1, rr1, sg1)

    def pair(t, carry):
        b = 2 * t + 2
        waitg(ra0, rr0, sg0)
        waitst(ob0, so0)
        comp_st(b, ra0, rr0, ob0, so0)
        gath(jnp.minimum(b + 2, NBB - 1), ra0, rr0, sg0)
        waitg(ra1, rr1, sg1)
        waitst(ob1, so1)
        comp_st(b + 1, ra1, rr1, ob1, so1)
        gath(jnp.minimum(b + 3, NBB - 1), ra1, rr1, sg1)
        return carry

    lax.fori_loop(0, (NBB - 3) // 2, pair, 0)
    waitg(ra0, rr0, sg0)
    waitst(ob0, so0)
    comp_st(NBB - 1, ra0, rr0, ob0, so0)
    waitg(ra1, rr1, sg1)
    waitst(ob0, so0)
    waitst(ob1, so1)


_bond_kernel = pl.kernel(
    _bond_body,
    out_type=jax.ShapeDtypeStruct((CHB, HID), jnp.float32),
    mesh=plsc.VectorSubcoreMesh(core_axis_name="c", subcore_axis_name="s"),
    scratch_types=[
        pltpu.VMEM((NBB, BB), jnp.int32),
        pltpu.VMEM((NBB, BB), jnp.int32),
        pltpu.VMEM((BB, HID), jnp.float32),
        pltpu.VMEM((BB, HID), jnp.float32),
        pltpu.VMEM((BB, HID), jnp.float32),
        pltpu.VMEM((BB, HID), jnp.float32),
        pltpu.VMEM((BB, HID), jnp.float32),
        pltpu.VMEM((BB, HID), jnp.float32),
        pltpu.SemaphoreType.DMA,
        pltpu.SemaphoreType.DMA,
        pltpu.SemaphoreType.DMA,
        pltpu.SemaphoreType.DMA,
    ],
)


def _mm_body(g_ref, in_ref, wt_ref, b_ref, o_ref):
    mm = jnp.dot(g_ref[...], wt_ref[...], preferred_element_type=jnp.float32)
    o_ref[...] = jnp.maximum(in_ref[...] + mm + b_ref[...], 0.0)


def _mm_body_acc(m_ref, g_ref, in_ref, wt_ref, b_ref, o_ref):
    mm = jnp.dot(g_ref[...], wt_ref[...], preferred_element_type=jnp.float32)
    o_ref[...] = jnp.maximum(in_ref[...] + mm + b_ref[...], 0.0)


def _linear_relu_chunk(j, m, g_j, input_bond, wt, b2d):
    grid = CHB // MM_BLK
    gspec = pl.BlockSpec((MM_BLK, HID), lambda i: (i, 0))
    inspec = pl.BlockSpec((MM_BLK, HID), lambda i, j=j: (j * grid + i, 0))
    wspec = pl.BlockSpec((HID, HID), lambda i: (0, 0))
    bspec = pl.BlockSpec((1, HID), lambda i: (0, 0))
    outspec = pl.BlockSpec((MM_BLK, HID), lambda i, j=j: (j * grid + i, 0))
    out_shape = jax.ShapeDtypeStruct((N_BONDS, HID), jnp.float32)
    if j == 0:
        return pl.pallas_call(
            _mm_body,
            grid=(grid,),
            in_specs=[gspec, inspec, wspec, bspec],
            out_specs=outspec,
            out_shape=out_shape,
        )(g_j, input_bond, wt, b2d)
    mspec = pl.BlockSpec((8, HID), lambda i: (0, 0))
    return pl.pallas_call(
        _mm_body_acc,
        grid=(grid,),
        in_specs=[mspec, gspec, inspec, wspec, bspec],
        out_specs=outspec,
        out_shape=out_shape,
        input_output_aliases={0: 0},
    )(m, g_j, input_bond, wt, b2d)


def kernel(message_atom, message_bond, a2b, b2a, b2revb, input_bond, W_bond, b_bond):
    a2b = a2b.astype(jnp.int32)
    b2a = b2a.astype(jnp.int32)
    b2revb = b2revb.astype(jnp.int32)

    ma_pad = jnp.pad(message_atom, ((0, PA - N_ATOMS), (0, 0)))
    pad_idx = jnp.arange((PA - N_ATOMS) * MAX_NB, dtype=jnp.int32) % N_BONDS
    a2b_pad = jnp.concatenate([a2b.reshape(-1), pad_idx])
    a2b_pad = a2b_pad.reshape(NW, 2 * NBA, 128)
    b2a_r = b2a.reshape(NCH, NW, NBB, BB)
    b2revb_r = b2revb.reshape(NCH, NW, NBB, BB)

    manew_pad = _atom_kernel(a2b_pad, ma_pad, message_bond)
    wt = W_bond.T
    b2d = b_bond.reshape(1, HID)
    mb = None
    for j in range(NCH):
        g_j = _bond_kernel(b2a_r[j], b2revb_r[j], manew_pad, message_bond)
        mb = _linear_relu_chunk(j, mb, g_j, input_bond, wt, b2d)
    return (manew_pad[:N_ATOMS], mb)
